# baseline (speedup 1.0000x reference)
import functools

import jax
import jax.numpy as jnp
from jax import lax
from jax.experimental import pallas as pl
from jax.experimental.pallas import tpu as pltpu
from jax.experimental.pallas import tpu_sc as plsc

_B, _H, _W = 8, 512, 512
_P = _B * _H * _W
_NC, _NS, _L = 2, 16, 16
_NW = _NC * _NS
_PIX_PER_W = _P // _NW
_CHUNK = 4096
_NCHUNK = _PIX_PER_W // _CHUNK
_GB = 128
_NGB = _CHUNK // _GB


def _floor_parts(g):
    t = g.astype(jnp.int32)
    tf = t.astype(jnp.float32)
    f = jnp.where(tf > g, tf - 1.0, tf)
    return f.astype(jnp.int32), g - f


def _sc_body(ta, tb, tc_, td, gx_hbm, gy_hbm, out_hbm,
             gxv, gyv, idxv, pav, pbv, pcv, pdv,
             wav, wbv, wcv, wdv, outv, sem0, sem1):
    wid = lax.axis_index("s") * _NC + lax.axis_index("c")
    iota = lax.iota(jnp.int32, _L)

    def compute_and_fire(c, q):
        base = wid * _PIX_PER_W + c * _CHUNK
        bbase = (base >> 18) << 18
        pltpu.sync_copy(gx_hbm.at[pl.ds(base, _CHUNK)], gxv)
        pltpu.sync_copy(gy_hbm.at[pl.ds(base, _CHUNK)], gyv)

        def idx_body(i, _):
            off = i * _L
            gx = gxv[pl.ds(off, _L)]
            gy = gyv[pl.ds(off, _L)]
            x0, fxr = _floor_parts(gx)
            y0, fyr = _floor_parts(gy)
            exr = 1.0 - fxr
            eyr = 1.0 - fyr
            wa = exr * eyr
            wb = fxr * eyr
            wc = exr * fyr
            wd = fxr * fyr
            zero = jnp.zeros_like(wa)
            mx = gx < 0.0
            wa = jnp.where(mx, wa + wb, wa)
            wb = jnp.where(mx, zero, wb)
            wc = jnp.where(mx, wc + wd, wc)
            wd = jnp.where(mx, zero, wd)
            my = gy < 0.0
            wa = jnp.where(my, wa + wc, wa)
            wc = jnp.where(my, zero, wc)
            wb = jnp.where(my, wb + wd, wb)
            wd = jnp.where(my, zero, wd)
            x0c = jnp.minimum(jnp.maximum(x0, 0), _W - 1)
            y0c = jnp.minimum(jnp.maximum(y0, 0), _H - 1)
            gidx = bbase + (y0c << 9) + x0c
            idxv[q, pl.ds(off, _L)] = gidx
            wav[q, pl.ds(off, _L)] = wa
            wbv[q, pl.ds(off, _L)] = wb
            wcv[q, pl.ds(off, _L)] = wc
            wdv[q, pl.ds(off, _L)] = wd
            return 0

        lax.fori_loop(0, _CHUNK // _L, idx_body, 0)

        sem = sem0 if q == 0 else sem1

        def fire(j, _):
            sl = pl.ds(j * _GB, _GB)
            isl = idxv.at[q].at[sl]
            pltpu.async_copy(ta.at[isl], pav.at[q].at[sl], sem)
            pltpu.async_copy(tb.at[isl], pbv.at[q].at[sl], sem)
            pltpu.async_copy(tc_.at[isl], pcv.at[q].at[sl], sem)
            pltpu.async_copy(td.at[isl], pdv.at[q].at[sl], sem)
            return 0

        lax.fori_loop(0, _NGB, fire, 0)

    def drain_and_blend(c, q):
        base = wid * _PIX_PER_W + c * _CHUNK
        sem = sem0 if q == 0 else sem1
        dummy = ta.at[pl.ds(0, _CHUNK)]
        pltpu.make_async_copy(dummy, pav.at[q], sem).wait()
        pltpu.make_async_copy(dummy, pbv.at[q], sem).wait()
        pltpu.make_async_copy(dummy, pcv.at[q], sem).wait()
        pltpu.make_async_copy(dummy, pdv.at[q], sem).wait()

        def blend_body(i, _):
            off = i * _L
            sl = pl.ds(off, _L)
            o = (wav[q, sl] * pav[q, sl] + wbv[q, sl] * pbv[q, sl]
                 + wcv[q, sl] * pcv[q, sl] + wdv[q, sl] * pdv[q, sl])
            outv[sl] = o
            return 0

        lax.fori_loop(0, _CHUNK // _L, blend_body, 0)
        pltpu.sync_copy(outv, out_hbm.at[pl.ds(base, _CHUNK)])

    compute_and_fire(0, 0)

    def chunk_pair(m, _):
        c = 2 * m + 1
        compute_and_fire(c, 1)
        drain_and_blend(c - 1, 0)
        compute_and_fire(c + 1, 0)
        drain_and_blend(c, 1)
        return 0

    lax.fori_loop(0, (_NCHUNK - 2) // 2, chunk_pair, 0)
    compute_and_fire(_NCHUNK - 1, 1)
    drain_and_blend(_NCHUNK - 2, 0)
    drain_and_blend(_NCHUNK - 1, 1)


_sc_call = functools.partial(
    pl.kernel,
    out_type=jax.ShapeDtypeStruct((_P,), jnp.float32),
    mesh=plsc.VectorSubcoreMesh(core_axis_name="c", subcore_axis_name="s",
                                num_cores=_NC, num_subcores=_NS),
    scratch_types=[
        pltpu.VMEM((_CHUNK,), jnp.float32),
        pltpu.VMEM((_CHUNK,), jnp.float32),
        pltpu.VMEM((2, _CHUNK), jnp.int32),
        pltpu.VMEM((2, _CHUNK), jnp.float32),
        pltpu.VMEM((2, _CHUNK), jnp.float32),
        pltpu.VMEM((2, _CHUNK), jnp.float32),
        pltpu.VMEM((2, _CHUNK), jnp.float32),
        pltpu.VMEM((2, _CHUNK), jnp.float32),
        pltpu.VMEM((2, _CHUNK), jnp.float32),
        pltpu.VMEM((2, _CHUNK), jnp.float32),
        pltpu.VMEM((2, _CHUNK), jnp.float32),
        pltpu.VMEM((_CHUNK,), jnp.float32),
        pltpu.SemaphoreType.DMA,
        pltpu.SemaphoreType.DMA,
    ],
)(_sc_body)


def kernel(src, flow):
    s = src[..., 0]
    sx = jnp.concatenate([s[:, :, 1:], s[:, :, -1:]], axis=2)
    sy = jnp.concatenate([s[:, 1:, :], s[:, -1:, :]], axis=1)
    sxy = jnp.concatenate([sx[:, 1:, :], sx[:, -1:, :]], axis=1)
    xs = jnp.arange(_W, dtype=jnp.float32)
    ys = jnp.arange(_H, dtype=jnp.float32)
    gx = (flow[..., 0] + xs[None, None, :]).reshape(_P)
    gy = (flow[..., 1] + ys[None, :, None]).reshape(_P)
    out = _sc_call(s.reshape(_P), sx.reshape(_P), sy.reshape(_P),
                   sxy.reshape(_P), gx, gy)
    return out.reshape(_B, _H, _W, 1)


# speedup vs baseline: 1.4821x; 1.4821x over previous
import functools

import jax
import jax.numpy as jnp
from jax import lax
from jax.experimental import pallas as pl
from jax.experimental.pallas import tpu as pltpu
from jax.experimental.pallas import tpu_sc as plsc

_B, _H, _W = 8, 512, 512
_P = _B * _H * _W
_NC, _NS, _L = 2, 16, 16
_NW = _NC * _NS
_PIX_PER_W = _P // _NW
_CHUNK = 4096
_NCHUNK = _PIX_PER_W // _CHUNK
_GB = 128
_NGB = _CHUNK // _GB


def _floor_parts(g):
    t = g.astype(jnp.int32)
    tf = t.astype(jnp.float32)
    f = jnp.where(tf > g, tf - 1.0, tf)
    return f.astype(jnp.int32), g - f


def _sc_body(ttop, tbot, gx_hbm, gy_hbm, out_hbm,
             gxv, gyv, idxv, ptv, pbv,
             wav, wbv, wcv, wdv, outv, tmpi, sem0, sem1):
    wid = lax.axis_index("s") * _NC + lax.axis_index("c")

    def compute_and_fire(c, q):
        base = wid * _PIX_PER_W + c * _CHUNK
        bbase = (base >> 18) << 18
        pltpu.sync_copy(gx_hbm.at[pl.ds(base, _CHUNK)], gxv)
        pltpu.sync_copy(gy_hbm.at[pl.ds(base, _CHUNK)], gyv)

        def idx_body(i, _):
            off = i * _L
            gx = gxv[pl.ds(off, _L)]
            gy = gyv[pl.ds(off, _L)]
            x0, fxr = _floor_parts(gx)
            y0, fyr = _floor_parts(gy)
            exr = 1.0 - fxr
            eyr = 1.0 - fyr
            wa = exr * eyr
            wb = fxr * eyr
            wc = exr * fyr
            wd = fxr * fyr
            zero = jnp.zeros_like(wa)
            mx = gx < 0.0
            wa = jnp.where(mx, wa + wb, wa)
            wb = jnp.where(mx, zero, wb)
            wc = jnp.where(mx, wc + wd, wc)
            wd = jnp.where(mx, zero, wd)
            my = gy < 0.0
            wa = jnp.where(my, wa + wc, wa)
            wc = jnp.where(my, zero, wc)
            wb = jnp.where(my, wb + wd, wb)
            wd = jnp.where(my, zero, wd)
            x0c = jnp.minimum(jnp.maximum(x0, 0), _W - 1)
            y0c = jnp.minimum(jnp.maximum(y0, 0), _H - 1)
            gidx = bbase + (y0c << 9) + x0c
            idxv[q, pl.ds(off, _L)] = gidx
            wav[q, pl.ds(off, _L)] = wa
            wbv[q, pl.ds(off, _L)] = wb
            wcv[q, pl.ds(off, _L)] = wc
            wdv[q, pl.ds(off, _L)] = wd
            return 0

        lax.fori_loop(0, _CHUNK // _L, idx_body, 0)

        sem = sem0 if q == 0 else sem1

        def fire(j, _):
            sl = pl.ds(j * _GB, _GB)
            isl = idxv.at[q].at[sl]
            pltpu.async_copy(ttop.at[isl], ptv.at[q].at[sl], sem)
            pltpu.async_copy(tbot.at[isl], pbv.at[q].at[sl], sem)
            return 0

        lax.fori_loop(0, _NGB, fire, 0)

    def drain_and_blend(c, q):
        base = wid * _PIX_PER_W + c * _CHUNK
        sem = sem0 if q == 0 else sem1
        dummy = ttop.at[pl.ds(0, _CHUNK)]
        pltpu.make_async_copy(dummy, ptv.at[q], sem).wait()
        pltpu.make_async_copy(dummy, pbv.at[q], sem).wait()
        tmpf = tmpi.bitcast(jnp.float32)

        def blend_body(i, _):
            off = i * _L
            sl = pl.ds(off, _L)
            pt = ptv[q, sl]
            pbt = pbv[q, sl]
            himask = jnp.full_like(pt, -65536)
            tsl = pl.ds(0, _L)
            tmpi[0, tsl] = pt << 16
            tmpi[1, tsl] = pt & himask
            tmpi[2, tsl] = pbt << 16
            tmpi[3, tsl] = pbt & himask
            o = (wav[q, sl] * tmpf[0, tsl] + wbv[q, sl] * tmpf[1, tsl]
                 + wcv[q, sl] * tmpf[2, tsl] + wdv[q, sl] * tmpf[3, tsl])
            outv[sl] = o
            return 0

        lax.fori_loop(0, _CHUNK // _L, blend_body, 0)
        pltpu.sync_copy(outv, out_hbm.at[pl.ds(base, _CHUNK)])

    compute_and_fire(0, 0)

    def chunk_pair(m, _):
        c = 2 * m + 1
        compute_and_fire(c, 1)
        drain_and_blend(c - 1, 0)
        compute_and_fire(c + 1, 0)
        drain_and_blend(c, 1)
        return 0

    lax.fori_loop(0, (_NCHUNK - 2) // 2, chunk_pair, 0)
    compute_and_fire(_NCHUNK - 1, 1)
    drain_and_blend(_NCHUNK - 2, 0)
    drain_and_blend(_NCHUNK - 1, 1)


_sc_call = functools.partial(
    pl.kernel,
    out_type=jax.ShapeDtypeStruct((_P,), jnp.float32),
    mesh=plsc.VectorSubcoreMesh(core_axis_name="c", subcore_axis_name="s",
                                num_cores=_NC, num_subcores=_NS),
    scratch_types=[
        pltpu.VMEM((_CHUNK,), jnp.float32),
        pltpu.VMEM((_CHUNK,), jnp.float32),
        pltpu.VMEM((2, _CHUNK), jnp.int32),
        pltpu.VMEM((2, _CHUNK), jnp.int32),
        pltpu.VMEM((2, _CHUNK), jnp.int32),
        pltpu.VMEM((2, _CHUNK), jnp.float32),
        pltpu.VMEM((2, _CHUNK), jnp.float32),
        pltpu.VMEM((2, _CHUNK), jnp.float32),
        pltpu.VMEM((2, _CHUNK), jnp.float32),
        pltpu.VMEM((_CHUNK,), jnp.float32),
        pltpu.VMEM((4, _L), jnp.int32),
        pltpu.SemaphoreType.DMA,
        pltpu.SemaphoreType.DMA,
    ],
)(_sc_body)


def kernel(src, flow):
    s = src[..., 0]
    sx = jnp.concatenate([s[:, :, 1:], s[:, :, -1:]], axis=2)
    sy = jnp.concatenate([s[:, 1:, :], s[:, -1:, :]], axis=1)
    sxy = jnp.concatenate([sx[:, 1:, :], sx[:, -1:, :]], axis=1)
    bf = jnp.bfloat16
    ttop = lax.bitcast_convert_type(
        jnp.stack([s.astype(bf), sx.astype(bf)], axis=-1), jnp.int32
    ).reshape(_P)
    tbot = lax.bitcast_convert_type(
        jnp.stack([sy.astype(bf), sxy.astype(bf)], axis=-1), jnp.int32
    ).reshape(_P)
    xs = jnp.arange(_W, dtype=jnp.float32)
    ys = jnp.arange(_H, dtype=jnp.float32)
    gx = (flow[..., 0] + xs[None, None, :]).reshape(_P)
    gy = (flow[..., 1] + ys[None, :, None]).reshape(_P)
    out = _sc_call(ttop, tbot, gx, gy)
    return out.reshape(_B, _H, _W, 1)


# speedup vs baseline: 1.6958x; 1.1442x over previous
import functools

import jax
import jax.numpy as jnp
from jax import lax
from jax.experimental import pallas as pl
from jax.experimental.pallas import tpu as pltpu
from jax.experimental.pallas import tpu_sc as plsc

_B, _H, _W = 8, 512, 512
_P = _B * _H * _W
_NC, _NS, _L = 2, 16, 16
_NW = _NC * _NS
_PIX_PER_W = _P // _NW
_CHUNK = 4096
_NCHUNK = _PIX_PER_W // _CHUNK
_GB = 128
_NGB = _CHUNK // _GB


def _floor_parts(g):
    t = g.astype(jnp.int32)
    tf = t.astype(jnp.float32)
    f = jnp.where(tf > g, tf - 1.0, tf)
    return f.astype(jnp.int32), g - f


def _sc_body(ttop, tbot, gx_hbm, gy_hbm, out_hbm,
             gxv, gyv, idxv, ptv, pbv,
             wav, wbv, wcv, wdv, outv, tmpi, sem0, sem1):
    wid = lax.axis_index("s") * _NC + lax.axis_index("c")

    def compute_and_fire(c, q):
        base = wid * _PIX_PER_W + c * _CHUNK
        bbase = (base >> 18) << 18
        pltpu.sync_copy(gx_hbm.at[pl.ds(base, _CHUNK)], gxv)
        pltpu.sync_copy(gy_hbm.at[pl.ds(base, _CHUNK)], gyv)

        @plsc.parallel_loop(0, _CHUNK // _L, 1, unroll=4)
        def idx_body(i):
            off = i * _L
            gx = gxv[pl.ds(off, _L)]
            gy = gyv[pl.ds(off, _L)]
            x0, fxr = _floor_parts(gx)
            y0, fyr = _floor_parts(gy)
            exr = 1.0 - fxr
            eyr = 1.0 - fyr
            wa = exr * eyr
            wb = fxr * eyr
            wc = exr * fyr
            wd = fxr * fyr
            zero = jnp.zeros_like(wa)
            mx = gx < 0.0
            wa = jnp.where(mx, wa + wb, wa)
            wb = jnp.where(mx, zero, wb)
            wc = jnp.where(mx, wc + wd, wc)
            wd = jnp.where(mx, zero, wd)
            my = gy < 0.0
            wa = jnp.where(my, wa + wc, wa)
            wc = jnp.where(my, zero, wc)
            wb = jnp.where(my, wb + wd, wb)
            wd = jnp.where(my, zero, wd)
            x0c = jnp.minimum(jnp.maximum(x0, 0), _W - 1)
            y0c = jnp.minimum(jnp.maximum(y0, 0), _H - 1)
            gidx = bbase + (y0c << 9) + x0c
            idxv[q, pl.ds(off, _L)] = gidx
            wav[q, pl.ds(off, _L)] = wa
            wbv[q, pl.ds(off, _L)] = wb
            wcv[q, pl.ds(off, _L)] = wc
            wdv[q, pl.ds(off, _L)] = wd

        sem = sem0 if q == 0 else sem1

        def fire(j, _):
            sl = pl.ds(j * _GB, _GB)
            isl = idxv.at[q].at[sl]
            pltpu.async_copy(ttop.at[isl], ptv.at[q].at[sl], sem)
            pltpu.async_copy(tbot.at[isl], pbv.at[q].at[sl], sem)
            return 0

        lax.fori_loop(0, _NGB, fire, 0)

    def drain_and_blend(c, q):
        base = wid * _PIX_PER_W + c * _CHUNK
        sem = sem0 if q == 0 else sem1
        dummy = ttop.at[pl.ds(0, _CHUNK)]
        pltpu.make_async_copy(dummy, ptv.at[q], sem).wait()
        pltpu.make_async_copy(dummy, pbv.at[q], sem).wait()
        tmpf = tmpi.bitcast(jnp.float32)

        @plsc.parallel_loop(0, _CHUNK // _L, 1, unroll=4)
        def blend_body(i):
            off = i * _L
            sl = pl.ds(off, _L)
            pt = ptv[q, sl]
            pbt = pbv[q, sl]
            himask = jnp.full_like(pt, -65536)
            tmpi[0, sl] = pt << 16
            tmpi[1, sl] = pt & himask
            tmpi[2, sl] = pbt << 16
            tmpi[3, sl] = pbt & himask
            o = (wav[q, sl] * tmpf[0, sl] + wbv[q, sl] * tmpf[1, sl]
                 + wcv[q, sl] * tmpf[2, sl] + wdv[q, sl] * tmpf[3, sl])
            outv[sl] = o
        pltpu.sync_copy(outv, out_hbm.at[pl.ds(base, _CHUNK)])

    compute_and_fire(0, 0)

    def chunk_pair(m, _):
        c = 2 * m + 1
        compute_and_fire(c, 1)
        drain_and_blend(c - 1, 0)
        compute_and_fire(c + 1, 0)
        drain_and_blend(c, 1)
        return 0

    lax.fori_loop(0, (_NCHUNK - 2) // 2, chunk_pair, 0)
    compute_and_fire(_NCHUNK - 1, 1)
    drain_and_blend(_NCHUNK - 2, 0)
    drain_and_blend(_NCHUNK - 1, 1)


_sc_call = functools.partial(
    pl.kernel,
    out_type=jax.ShapeDtypeStruct((_P,), jnp.float32),
    mesh=plsc.VectorSubcoreMesh(core_axis_name="c", subcore_axis_name="s",
                                num_cores=_NC, num_subcores=_NS),
    scratch_types=[
        pltpu.VMEM((_CHUNK,), jnp.float32),
        pltpu.VMEM((_CHUNK,), jnp.float32),
        pltpu.VMEM((2, _CHUNK), jnp.int32),
        pltpu.VMEM((2, _CHUNK), jnp.int32),
        pltpu.VMEM((2, _CHUNK), jnp.int32),
        pltpu.VMEM((2, _CHUNK), jnp.float32),
        pltpu.VMEM((2, _CHUNK), jnp.float32),
        pltpu.VMEM((2, _CHUNK), jnp.float32),
        pltpu.VMEM((2, _CHUNK), jnp.float32),
        pltpu.VMEM((_CHUNK,), jnp.float32),
        pltpu.VMEM((4, _CHUNK), jnp.int32),
        pltpu.SemaphoreType.DMA,
        pltpu.SemaphoreType.DMA,
    ],
)(_sc_body)


def kernel(src, flow):
    s = src[..., 0]
    sx = jnp.concatenate([s[:, :, 1:], s[:, :, -1:]], axis=2)
    sy = jnp.concatenate([s[:, 1:, :], s[:, -1:, :]], axis=1)
    sxy = jnp.concatenate([sx[:, 1:, :], sx[:, -1:, :]], axis=1)
    bf = jnp.bfloat16
    ttop = lax.bitcast_convert_type(
        jnp.stack([s.astype(bf), sx.astype(bf)], axis=-1), jnp.int32
    ).reshape(_P)
    tbot = lax.bitcast_convert_type(
        jnp.stack([sy.astype(bf), sxy.astype(bf)], axis=-1), jnp.int32
    ).reshape(_P)
    xs = jnp.arange(_W, dtype=jnp.float32)
    ys = jnp.arange(_H, dtype=jnp.float32)
    gx = (flow[..., 0] + xs[None, None, :]).reshape(_P)
    gy = (flow[..., 1] + ys[None, :, None]).reshape(_P)
    out = _sc_call(ttop, tbot, gx, gy)
    return out.reshape(_B, _H, _W, 1)
